# Initial kernel scaffold; baseline (speedup 1.0000x reference)
import jax
import jax.numpy as jnp
from jax.experimental import pallas as pl


def kernel(node_x, eb_nodes_map, eb_batch, nb_edges_map, nb_batch, bn_gamma, bn_beta, Wk0, Wv0, seed0, ff0w, ff0b, Wk1, Wv1, seed1, ff1w, ff1b, Wc, bc):
    raise NotImplementedError("write your pallas kernel here")


# speedup vs baseline: 53.0285x; 53.0285x over previous
import functools

import jax
import jax.numpy as jnp
import numpy as np
from jax import lax
from jax.experimental import pallas as pl
from jax.experimental.pallas import tpu as pltpu
from jax.experimental.pallas import tpu_sc as plsc

N = 10000
E = 10000
M = 320000
D = 128
HEADS = 4
DH = 32

ROW = 256
OROW = 576
NW = 32
MT = M // NW
K = 128
PADLEN = 2 * K
SEG_PAD = 16
DPAD = 2 * E
NSTAGE = 4


def _score_table(x, wk, seed):
  cols = []
  for h_ in range(HEADS):
    ws_h = jnp.sum(wk[:, DH * h_:DH * (h_ + 1)] * seed[h_:h_ + 1, :],
                   axis=1, keepdims=True)
    cols.append(ws_h)
  ws = jnp.concatenate(cols, axis=1) * (1.0 / np.sqrt(DH))
  score = jnp.dot(x, ws, preferred_element_type=jnp.float32)
  gmax = jnp.max(score, axis=0, keepdims=True)
  et = jnp.exp(score - gmax)
  etb = jnp.concatenate(
      [jnp.broadcast_to(et[:, h_:h_ + 1], (x.shape[0], 16))
       for h_ in range(HEADS)], axis=1)
  return etb


def _prep0_body(x_ref, g_ref, b_ref, wk_ref, seed_ref, t_ref):
  gamma = g_ref[...] * (1.0 / np.sqrt(1.0 + 1e-5))
  h = x_ref[...] * gamma[None, :] + b_ref[...][None, :]
  etb = _score_table(h, wk_ref[...], seed_ref[...])
  t_ref[...] = jnp.concatenate(
      [h, etb, jnp.zeros((h.shape[0], 64), jnp.float32)], axis=1)


def _pooled_ff_ln(bd, wv, ffw, ffb):
  parts = []
  for h_ in range(HEADS):
    den = bd[:, 512 + 16 * h_:512 + 16 * h_ + 1] + 1e-12
    a = bd[:, D * h_:D * (h_ + 1)] / den
    parts.append(jnp.dot(a, wv[:, DH * h_:DH * (h_ + 1)],
                         preferred_element_type=jnp.float32))
  pooled = jnp.concatenate(parts, axis=1)
  ff = jnp.dot(pooled, ffw, preferred_element_type=jnp.float32) + ffb[None, :]
  h1 = pooled + jnp.maximum(ff, 0.0)
  mu = jnp.mean(h1, axis=1, keepdims=True)
  var = jnp.mean((h1 - mu) ** 2, axis=1, keepdims=True)
  y = (h1 - mu) / jnp.sqrt(var + 1e-5)
  return jnp.maximum(y, 0.0)


def _mid_body(bd_ref, wv_ref, ffw_ref, ffb_ref, wk_ref, seed_ref, t_ref):
  edge = _pooled_ff_ln(bd_ref[...], wv_ref[...], ffw_ref[...], ffb_ref[...])
  etb = _score_table(edge, wk_ref[...], seed_ref[...])
  t_ref[...] = jnp.concatenate(
      [edge, etb, jnp.zeros((edge.shape[0], 64), jnp.float32)], axis=1)


def _final_body(bd_ref, wv_ref, ffw_ref, ffb_ref, wc_ref, bc_ref, o_ref):
  node_h = _pooled_ff_ln(bd_ref[...], wv_ref[...], ffw_ref[...], ffb_ref[...])
  logits = jnp.dot(node_h, wc_ref[...],
                   preferred_element_type=jnp.float32) + bc_ref[...][None, :]
  m = jnp.max(logits, axis=1, keepdims=True)
  lse = m + jnp.log(jnp.sum(jnp.exp(logits - m), axis=1, keepdims=True))
  o_ref[...] = logits - lse


def _tc(body, out_shape, *args):
  return pl.pallas_call(
      body, out_shape=jax.ShapeDtypeStruct(out_shape, jnp.float32))(*args)


def _sc_body(t_ref, src_ref, dst_ref, bnd_ref, out_ref,
             idx_v, rows_v, accr_v, zrow_v, dtmp_v, btmp_v, shr_v, shrb_v,
             dst_s, bnd_s, gsem, ssem):
  sid = lax.axis_index("s")
  wid = sid * 2 + lax.axis_index("c")
  pltpu.sync_copy(bnd_ref.at[wid], btmp_v)
  pltpu.sync_copy(btmp_v, shrb_v.at[sid, pl.ds(0, 16)])
  pltpu.sync_copy(shrb_v.at[sid, pl.ds(0, 16)], bnd_s)
  lo = bnd_s[0]
  hi = bnd_s[1]
  m0 = wid * MT

  zeros16 = jnp.zeros((16,), jnp.float32)
  for i in range(OROW // 16):
    zrow_v[pl.ds(i * 16, 16)] = zeros16
  for r in range(NSTAGE):
    for i in range(OROW // 16):
      accr_v[r, pl.ds(i * 16, 16)] = zeros16

  def gap_fill(lo_g, hi_g):
    def gap(g, c):
      pltpu.sync_copy(zrow_v, out_ref.at[g])
      return c
    lax.fori_loop(lo_g, hi_g, gap, 0)

  def flush(args, d):
    cur, fcount, rb = args
    emit = cur >= 0
    f2 = jnp.where(emit, fcount + 1, fcount)
    rb2 = jnp.where(emit, lax.rem(rb + 1, NSTAGE), rb)

    @pl.when(emit)
    def _():
      pltpu.async_copy(accr_v.at[rb], out_ref.at[cur], ssem.at[rb])

      @pl.when(f2 >= NSTAGE)
      def _():
        pltpu.make_async_copy(
            accr_v.at[rb2], out_ref.at[0], ssem.at[rb2]).wait()

      for i in range(OROW // 16):
        accr_v[rb2, pl.ds(i * 16, 16)] = zeros16

    gap_fill(jnp.maximum(cur + 1, lo), d)
    return d, f2, rb2

  def elem(j, ec):
    cur, fcount, rb = ec
    d = dst_s[j]
    act = jnp.logical_and(d >= lo, d < hi)
    newseg = jnp.logical_and(act, d != cur)
    cur, fcount, rb = lax.cond(
        newseg, lambda a: flush(a, d), lambda a: a, (cur, fcount, rb))
    actf = jnp.where(act, jnp.float32(1.0), jnp.float32(0.0))
    row = [rows_v[j, pl.ds(k * 16, 16)] for k in range(8)]
    for h_ in range(HEADS):
      w = rows_v[j, pl.ds(D + h_ * 16, 16)] * actf
      for k in range(8):
        plsc.addupdate(accr_v.at[rb, pl.ds((h_ * 8 + k) * 16, 16)],
                       w * row[k])
      plsc.addupdate(accr_v.at[rb, pl.ds((32 + h_) * 16, 16)], w)
    return cur, fcount, rb

  def chunk(carry):
    jb, cur, fcount, rb, stop = carry
    jba = pl.multiple_of(jb, 16)
    pltpu.sync_copy(src_ref.at[pl.ds(jba, K)], idx_v)
    pltpu.sync_copy(dst_ref.at[pl.ds(jba, K)], dtmp_v)
    pltpu.sync_copy(dtmp_v, shr_v.at[sid])
    pltpu.sync_copy(shr_v.at[sid], dst_s)
    pltpu.async_copy(t_ref.at[idx_v], rows_v, gsem).wait()
    cur, fcount, rb = lax.fori_loop(0, K, elem, (cur, fcount, rb))
    stop = dst_s[K - 1] >= hi
    return jb + K, cur, fcount, rb, stop

  max_chunks = (M + PADLEN) // K

  def maybe_chunk(c, carry):
    del c
    live = jnp.logical_and(jnp.logical_not(carry[4]),
                           carry[0] + K <= M + PADLEN)
    return lax.cond(live, chunk, lambda x: x, carry)

  init = (m0, jnp.int32(-1), jnp.int32(0), jnp.int32(0), jnp.bool_(False))
  jb, cur, fcount, rb, stop = lax.fori_loop(0, max_chunks, maybe_chunk, init)

  _, fcount, _ = flush((cur, fcount, rb), hi)

  for r in range(NSTAGE):
    dr = lax.rem(fcount - 1 - r + 2 * NSTAGE, NSTAGE)
    i_r = fcount - dr
    @pl.when(jnp.logical_and(i_r >= 1, dr <= NSTAGE - 2))
    def _():
      pltpu.make_async_copy(accr_v.at[r], out_ref.at[0], ssem.at[r]).wait()


@functools.partial(
    pl.kernel,
    out_type=jax.ShapeDtypeStruct((E + SEG_PAD, OROW), jnp.float32),
    mesh=plsc.VectorSubcoreMesh(core_axis_name="c", subcore_axis_name="s"),
    scratch_types=[
        pltpu.VMEM((K,), jnp.int32),
        pltpu.VMEM((K, ROW), jnp.float32),
        pltpu.VMEM((NSTAGE, OROW), jnp.float32),
        pltpu.VMEM((OROW,), jnp.float32),
        pltpu.VMEM((K,), jnp.int32),
        pltpu.VMEM((16,), jnp.int32),
        pltpu.VMEM_SHARED((16, K), jnp.int32),
        pltpu.VMEM_SHARED((16, 128), jnp.int32),
        pltpu.SMEM((K,), jnp.int32),
        pltpu.SMEM((16,), jnp.int32),
        pltpu.SemaphoreType.DMA,
        pltpu.SemaphoreType.DMA((NSTAGE,)),
    ],
)
def _sc_kernel(t_ref, src_ref, dst_ref, bnd_ref, out_ref,
               idx_v, rows_v, accr_v, zrow_v, dtmp_v, btmp_v, shr_v, shrb_v,
               dst_s, bnd_s, gsem, ssem):
  _sc_body(t_ref, src_ref, dst_ref, bnd_ref, out_ref,
           idx_v, rows_v, accr_v, zrow_v, dtmp_v, btmp_v, shr_v, shrb_v,
           dst_s, bnd_s, gsem, ssem)


def _sc_pass(table, src, dst):
  srcp = jnp.concatenate([src, jnp.zeros((PADLEN,), jnp.int32)])
  dstp = jnp.concatenate([dst, jnp.full((PADLEN,), DPAD, jnp.int32)])
  ends = dst[(np.arange(1, NW) * MT) - 1] + 1
  lo = jnp.concatenate([jnp.zeros((1,), jnp.int32), ends])
  hi = jnp.concatenate([ends, jnp.full((1,), E + SEG_PAD, jnp.int32)])
  bnd = jnp.zeros((NW, 16), jnp.int32).at[:, 0].set(lo).at[:, 1].set(hi)
  return _sc_kernel(table, srcp, dstp, bnd)


def kernel(node_x, eb_nodes_map, eb_batch, nb_edges_map, nb_batch,
           bn_gamma, bn_beta, Wk0, Wv0, seed0, ff0w, ff0b,
           Wk1, Wv1, seed1, ff1w, ff1b, Wc, bc):
  src0 = eb_nodes_map.astype(jnp.int32)
  dst0 = eb_batch.astype(jnp.int32)
  src1 = nb_edges_map.astype(jnp.int32)
  dst1 = nb_batch.astype(jnp.int32)

  t0 = _tc(_prep0_body, (N, ROW), node_x, bn_gamma, bn_beta, Wk0, seed0)
  bd0 = _sc_pass(t0, src0, dst0)
  t1 = _tc(_mid_body, (E, ROW), bd0[:E], Wv0, ff0w, ff0b, Wk1, seed1)
  bd1 = _sc_pass(t1, src1, dst1)
  return _tc(_final_body, (N, 10), bd1[:N], Wv1, ff1w, ff1b, Wc, bc)
